# Initial kernel scaffold; baseline (speedup 1.0000x reference)
import jax
import jax.numpy as jnp
from jax.experimental import pallas as pl


def kernel(inputs, e, edge_index, W1, b1, W2, b2, W3, b3, W4, b4):
    raise NotImplementedError("write your pallas kernel here")


# speedup vs baseline: 3.5758x; 3.5758x over previous
import functools

import jax
import jax.numpy as jnp
from jax import lax
from jax.experimental import pallas as pl
from jax.experimental.pallas import tpu as pltpu
from jax.experimental.pallas import tpu_sc as plsc

N = 10000
D = 128
NC = 2
NS = 16
L = 16
CHUNK = 1024
STAGE = 256
SUB = 128
NSUB = CHUNK // SUB
NSTAGE = STAGE // SUB
ROWS_PER_TILE = 640
NPAD = NS * ROWS_PER_TILE


def _sc_aggregate_body(x_hbm, gidx_hbm, sidx_hbm, e_hbm, zeros_hbm, out_hbm,
                       gi_v, si_v, e_v, rows_v, acc_sh, sem):
  c = lax.axis_index("c")
  s = lax.axis_index("s")
  epad = e_hbm.shape[0]
  tile_edges = epad // NS
  nchunks = tile_edges // CHUNK

  r0 = s * ROWS_PER_TILE
  pltpu.sync_copy(zeros_hbm.at[pl.ds(0, ROWS_PER_TILE)],
                  acc_sh.at[pl.ds(r0, ROWS_PER_TILE)])
  plsc.subcore_barrier()

  tile_edge0 = s * tile_edges
  idxrow0 = (c * epad + tile_edge0) // SUB

  def chunk_body(i, carry):
    row_off = pl.multiple_of(idxrow0 + i * NSUB, NSUB)
    e_off = pl.multiple_of(tile_edge0 + i * CHUNK, CHUNK)
    pltpu.sync_copy(gidx_hbm.at[pl.ds(row_off, NSUB)], gi_v)
    pltpu.sync_copy(sidx_hbm.at[pl.ds(row_off, NSUB)], si_v)
    pltpu.sync_copy(e_hbm.at[pl.ds(e_off, CHUNK)], e_v)
    for h in range(CHUNK // STAGE):
      cps = [pltpu.async_copy(x_hbm.at[gi_v.at[h * NSTAGE + j]],
                              rows_v.at[pl.ds(j * SUB, SUB)], sem)
             for j in range(NSTAGE)]
      for cp in cps:
        cp.wait()

      def scale_body(g, carry2):
        base = pl.multiple_of(h * STAGE + g * L, L)
        ev16 = e_v[pl.ds(base, L)]
        for l in range(L):
          ev = jnp.full((L,), ev16[l], dtype=jnp.float32)
          k = g * L + l
          for q in range(D // L):
            sl = pl.ds(q * L, L)
            rows_v[k, sl] = rows_v[k, sl] * ev
        return carry2

      lax.fori_loop(0, STAGE // L, scale_body, 0)

      for j in range(NSTAGE):
        pltpu.sync_copy(rows_v.at[pl.ds(j * SUB, SUB)],
                        acc_sh.at[si_v.at[h * NSTAGE + j]], add=True)
    return carry

  lax.fori_loop(0, nchunks, chunk_body, 0)

  plsc.subcore_barrier()
  pltpu.sync_copy(acc_sh.at[pl.ds(r0, ROWS_PER_TILE)],
                  out_hbm.at[pl.ds(c * NPAD + r0, ROWS_PER_TILE)])


def _sc_aggregate(x, gidx, sidx, e_pad, zeros):
  mesh = plsc.VectorSubcoreMesh(core_axis_name="c", subcore_axis_name="s")
  fn = pl.kernel(
      _sc_aggregate_body,
      out_type=jax.ShapeDtypeStruct((2 * NPAD, D), jnp.float32),
      mesh=mesh,
      scratch_types=[
          pltpu.VMEM((NSUB, SUB), jnp.int32),
          pltpu.VMEM((NSUB, SUB), jnp.int32),
          pltpu.VMEM((CHUNK,), jnp.float32),
          pltpu.VMEM((STAGE, D), jnp.float32),
          pltpu.VMEM_SHARED((NPAD, D), jnp.float32),
          pltpu.SemaphoreType.DMA,
      ],
  )
  return fn(x, gidx, sidx, e_pad, zeros)


def _mlp_body(mi_ref, mo_ref, x_ref, w1a, w1b, w1c, b1, w2, b2, w3, b3,
              w4, b4, out_ref):
  dot = functools.partial(jnp.dot, preferred_element_type=jnp.float32)
  h = jnp.tanh(dot(mi_ref[...], w1a[...]) + dot(mo_ref[...], w1b[...])
               + dot(x_ref[...], w1c[...]) + b1[...])
  h = jnp.tanh(dot(h, w2[...]) + b2[...])
  h = jnp.tanh(dot(h, w3[...]) + b3[...])
  h = jnp.tanh(dot(h, w4[...]) + b4[...])
  out_ref[...] = h


def _mlp(mi, mo, x, W1, b1, W2, b2, W3, b3, W4, b4):
  blk = 2000
  grid = N // blk
  full = lambda shape: pl.BlockSpec(shape, lambda i: (0, 0))
  rows = pl.BlockSpec((blk, D), lambda i: (i, 0))
  return pl.pallas_call(
      _mlp_body,
      grid=(grid,),
      in_specs=[rows, rows, rows,
                full((D, D)), full((D, D)), full((D, D)), full((1, D)),
                full((D, D)), full((1, D)), full((D, D)), full((1, D)),
                full((D, D)), full((1, D))],
      out_specs=pl.BlockSpec((blk, D), lambda i: (i, 0)),
      out_shape=jax.ShapeDtypeStruct((N, D), jnp.float32),
  )(mi, mo, x, W1[:D], W1[D:2 * D], W1[2 * D:], b1.reshape(1, D),
    W2, b2.reshape(1, D), W3, b3.reshape(1, D), W4, b4.reshape(1, D))


def kernel(inputs, e, edge_index, W1, b1, W2, b2, W3, b3, W4, b4):
  x = inputs
  src = edge_index[0].astype(jnp.int32)
  dst = edge_index[1].astype(jnp.int32)
  E = src.shape[0]
  epad = -(-E // (NS * CHUNK)) * (NS * CHUNK)
  pad = epad - E
  ef = jnp.pad(e.reshape(-1), (0, pad))
  srcp = jnp.pad(src, (0, pad))
  dstp = jnp.pad(dst, (0, pad))
  gidx = jnp.concatenate([srcp, dstp]).reshape(-1, SUB)
  sidx = jnp.concatenate([dstp, srcp]).reshape(-1, SUB)
  zeros = jnp.zeros((ROWS_PER_TILE, D), jnp.float32)
  agg = _sc_aggregate(x, gidx, sidx, ef, zeros)
  mi = agg[:N]
  mo = agg[NPAD:NPAD + N]
  return _mlp(mi, mo, x, W1, b1, W2, b2, W3, b3, W4, b4)
